# Initial kernel scaffold; baseline (speedup 1.0000x reference)
import jax
import jax.numpy as jnp
from jax.experimental import pallas as pl


def kernel(edge_index, lg_edge_index, atom_feats, bondlength, cos_angles, timesteps, params):
    raise NotImplementedError("write your pallas kernel here")


# speedup vs baseline: 1.1272x; 1.1272x over previous
import math

import jax
import jax.numpy as jnp
from jax.experimental import pallas as pl

HID = 96
EMB = 64


def _linear(p, x):
    return x @ p["w"] + p["b"]


def _lnorm(p, x):
    m = x.mean(axis=-1, keepdims=True)
    v = ((x - m) ** 2).mean(axis=-1, keepdims=True)
    return (x - m) / jnp.sqrt(v + 1e-5) * p["g"] + p["b"]


def _mlp(p, x):
    return jax.nn.silu(_lnorm(p["ln"], _linear(p["lin"], x)))


def _rbf(x, vmin, vmax, bins):
    centers = jnp.linspace(vmin, vmax, bins)
    gamma = 1.0 / (centers[1] - centers[0]) ** 2
    return jnp.exp(-gamma * (x[:, None] - centers[None, :]) ** 2)


def _ts_embed(ts, dim):
    half = dim // 2
    f = math.log(10000.0) / (half - 1)
    freqs = jnp.exp(jnp.arange(half, dtype=jnp.float32) * -f)
    a = ts[:, None] * freqs[None, :]
    return jnp.concatenate([jnp.sin(a), jnp.cos(a)], axis=1)


def _egc(p, src, dst, n, x, y, t):
    tp = _linear(p["time_proj"], t)
    e_src = _linear(p["src_gate"], x) + tp
    e_dst = _linear(p["dst_gate"], x)
    m = e_src[src] + e_dst[dst] + _linear(p["edge_gate"], y)
    sigma = jax.nn.sigmoid(m)
    bh = _linear(p["dst_update"], x)
    sum_sigma_h = jax.ops.segment_sum(sigma * bh[src], dst, num_segments=n)
    sum_sigma = jax.ops.segment_sum(sigma, dst, num_segments=n)
    h = sum_sigma_h / (sum_sigma + 1e-6)
    xo = _linear(p["src_update"], x) + h
    xo = jax.nn.silu(_lnorm(p["ln_n"], xo))
    yo = jax.nn.silu(_lnorm(p["ln_e"], m))
    return x + xo, y + yo


def _readout_kernel(x_ref, w_ref, b_ref, o_ref):
    o_ref[...] = x_ref[...] @ w_ref[...] + b_ref[...]


def _readout(p, x):
    rows = x.shape[0]
    blk = 4096
    pad = (-rows) % blk
    xp = jnp.pad(x, ((0, pad), (0, 0)))
    out = pl.pallas_call(
        _readout_kernel,
        grid=((rows + pad) // blk,),
        in_specs=[
            pl.BlockSpec((blk, HID), lambda i: (i, 0)),
            pl.BlockSpec((HID, 1), lambda i: (0, 0)),
            pl.BlockSpec((1,), lambda i: (0,)),
        ],
        out_specs=pl.BlockSpec((blk, 1), lambda i: (i, 0)),
        out_shape=jax.ShapeDtypeStruct((rows + pad, 1), jnp.float32),
    )(xp, p["w"], p["b"])
    return out[:rows]


def kernel(edge_index, lg_edge_index, atom_feats, bondlength, cos_angles, timesteps, params):
    src, dst = edge_index[0], edge_index[1]
    lsrc, ldst = lg_edge_index[0], lg_edge_index[1]
    n = atom_feats.shape[0]
    e = bondlength.shape[0]
    t = _ts_embed(timesteps, EMB)
    t = _mlp(params["time_emb"][0], t)
    t = _mlp(params["time_emb"][1], t)
    x = _mlp(params["atom_emb"], atom_feats)
    y = _rbf(bondlength, 0.0, 8.0, 80)
    y = _mlp(params["edge_emb"][0], y)
    y = _mlp(params["edge_emb"][1], y)
    z = _rbf(cos_angles, -1.0, 1.0, 40)
    z = _mlp(params["angle_emb"][0], z)
    z = _mlp(params["angle_emb"][1], z)
    for lp in params["alignn"]:
        x, m = _egc(lp["node"], src, dst, n, x, y, t)
        y, z = _egc(lp["edge"], lsrc, ldst, e, m, z, t)
    for lp in params["gcn"]:
        x, y = _egc(lp, src, dst, n, x, y, t)
    xe, ye = _egc(params["edges_l1"], src, dst, n, x, y, t)
    xe, ye = _egc(params["edges_l2"], src, dst, n, xe, ye, t)
    edge_out = _readout(params["edges_ro"], ye)
    xa, ya = _egc(params["atoms_l"], src, dst, n, x, y, t)
    atom_out = _readout(params["atoms_ro"], xa)
    return jnp.concatenate([atom_out, edge_out], axis=0)


# speedup vs baseline: 1.2114x; 1.0747x over previous
import functools
import math

import jax
import jax.numpy as jnp
from jax import lax
from jax.experimental import pallas as pl
from jax.experimental.pallas import tpu as pltpu, tpu_sc as plsc

HID = 96
EMB = 64

N_NODES = 50000
N_EDGES = 800000
BLK_N = 2000
BLK_E = 3200


def _ln_silu(h, g, b):
    m = h.mean(axis=-1, keepdims=True)
    v = ((h - m) ** 2).mean(axis=-1, keepdims=True)
    h = (h - m) / jnp.sqrt(v + 1e-5) * g + b
    return h * jax.nn.sigmoid(h)


def _emb2_kernel(xs_ref, w1_ref, b1_ref, g1_ref, n1_ref, w2_ref, b2_ref,
                 g2_ref, n2_ref, o_ref, *, vmin, vmax, bins):
    xs = xs_ref[...]
    delta = (vmax - vmin) / (bins - 1)
    centers = vmin + delta * lax.broadcasted_iota(jnp.int32, (1, bins), 1).astype(jnp.float32)
    gamma = 1.0 / (delta * delta)
    r = jnp.exp(-gamma * (xs - centers) ** 2)
    h = _ln_silu(r @ w1_ref[...] + b1_ref[...], g1_ref[...], n1_ref[...])
    h = _ln_silu(h @ w2_ref[...] + b2_ref[...], g2_ref[...], n2_ref[...])
    o_ref[...] = h


def _emb2(xs, p1, p2, vmin, vmax, bins, blk):
    rows = xs.shape[0]
    d1 = p1["lin"]["w"].shape[1]
    d2 = p2["lin"]["w"].shape[1]
    f = pl.pallas_call(
        functools.partial(_emb2_kernel, vmin=vmin, vmax=vmax, bins=bins),
        grid=(rows // blk,),
        in_specs=[
            pl.BlockSpec((blk, 1), lambda i: (i, 0)),
            pl.BlockSpec((bins, d1), lambda i: (0, 0)),
            pl.BlockSpec((1, d1), lambda i: (0, 0)),
            pl.BlockSpec((1, d1), lambda i: (0, 0)),
            pl.BlockSpec((1, d1), lambda i: (0, 0)),
            pl.BlockSpec((d1, d2), lambda i: (0, 0)),
            pl.BlockSpec((1, d2), lambda i: (0, 0)),
            pl.BlockSpec((1, d2), lambda i: (0, 0)),
            pl.BlockSpec((1, d2), lambda i: (0, 0)),
        ],
        out_specs=pl.BlockSpec((blk, d2), lambda i: (i, 0)),
        out_shape=jax.ShapeDtypeStruct((rows, d2), jnp.float32),
    )
    r2 = lambda a: a.reshape(1, -1)
    return f(xs[:, None], p1["lin"]["w"], r2(p1["lin"]["b"]), r2(p1["ln"]["g"]),
             r2(p1["ln"]["b"]), p2["lin"]["w"], r2(p2["lin"]["b"]),
             r2(p2["ln"]["g"]), r2(p2["ln"]["b"]))


def _atom_emb_kernel(x_ref, w_ref, b_ref, g_ref, n_ref, o_ref):
    h = x_ref[...] @ w_ref[...] + b_ref[...]
    o_ref[...] = _ln_silu(h, g_ref[...], n_ref[...])


def _atom_emb(x, p):
    rows, din = x.shape
    f = pl.pallas_call(
        _atom_emb_kernel,
        grid=(rows // BLK_N,),
        in_specs=[
            pl.BlockSpec((BLK_N, din), lambda i: (i, 0)),
            pl.BlockSpec((din, HID), lambda i: (0, 0)),
            pl.BlockSpec((1, HID), lambda i: (0, 0)),
            pl.BlockSpec((1, HID), lambda i: (0, 0)),
            pl.BlockSpec((1, HID), lambda i: (0, 0)),
        ],
        out_specs=pl.BlockSpec((BLK_N, HID), lambda i: (i, 0)),
        out_shape=jax.ShapeDtypeStruct((rows, HID), jnp.float32),
    )
    r2 = lambda a: a.reshape(1, -1)
    return f(x, p["lin"]["w"], r2(p["lin"]["b"]), r2(p["ln"]["g"]), r2(p["ln"]["b"]))


def _time_kernel(ts_ref, w1_ref, b1_ref, g1_ref, n1_ref, w2_ref, b2_ref,
                 g2_ref, n2_ref, wp_ref, bp_ref, o_ref):
    ts = ts_ref[...]
    half = EMB // 2
    fr = math.log(10000.0) / (half - 1)
    freqs = jnp.exp(lax.broadcasted_iota(jnp.int32, (1, half), 1).astype(jnp.float32) * -fr)
    a = ts * freqs
    t = jnp.concatenate([jnp.sin(a), jnp.cos(a)], axis=1)
    t = _ln_silu(t @ w1_ref[...] + b1_ref[...], g1_ref[...], n1_ref[...])
    t = _ln_silu(t @ w2_ref[...] + b2_ref[...], g2_ref[...], n2_ref[...])
    o_ref[...] = t @ wp_ref[...] + bp_ref[...]


def _time_tp(timesteps, params, n_layers_tp, wp_all, bp_all):
    p1, p2 = params["time_emb"]
    ts8 = jnp.zeros((8, 1), jnp.float32).at[0, 0].set(timesteps[0])
    r2 = lambda a: a.reshape(1, -1)
    f = pl.pallas_call(
        _time_kernel,
        out_shape=jax.ShapeDtypeStruct((8, n_layers_tp * HID), jnp.float32),
    )
    out = f(ts8, p1["lin"]["w"], r2(p1["lin"]["b"]), r2(p1["ln"]["g"]), r2(p1["ln"]["b"]),
            p2["lin"]["w"], r2(p2["lin"]["b"]), r2(p2["ln"]["g"]), r2(p2["ln"]["b"]),
            wp_all, r2(bp_all))
    return out[0].reshape(n_layers_tp, HID)


def _pre_kernel(x_ref, w_ref, b_ref, esrc_ref, edst_ref, bh_ref, xu_ref):
    r = x_ref[...] @ w_ref[...] + b_ref[...]
    blk = r.shape[0]
    z = jnp.zeros((blk, 128 - HID), jnp.float32)
    esrc_ref[...] = jnp.concatenate([r[:, 0:96], z], axis=1)
    edst_ref[...] = jnp.concatenate([r[:, 96:192], z], axis=1)
    bh_ref[...] = jnp.concatenate([r[:, 192:288], z], axis=1)
    xu_ref[...] = r[:, 288:384]


def _egc_pre(x, wcat, bcat, blk):
    rows = x.shape[0]
    f = pl.pallas_call(
        _pre_kernel,
        grid=(rows // blk,),
        in_specs=[
            pl.BlockSpec((blk, HID), lambda i: (i, 0)),
            pl.BlockSpec((HID, 384), lambda i: (0, 0)),
            pl.BlockSpec((1, 384), lambda i: (0, 0)),
        ],
        out_specs=[
            pl.BlockSpec((blk, 128), lambda i: (i, 0)),
            pl.BlockSpec((blk, 128), lambda i: (i, 0)),
            pl.BlockSpec((blk, 128), lambda i: (i, 0)),
            pl.BlockSpec((blk, HID), lambda i: (i, 0)),
        ],
        out_shape=[
            jax.ShapeDtypeStruct((rows, 128), jnp.float32),
            jax.ShapeDtypeStruct((rows, 128), jnp.float32),
            jax.ShapeDtypeStruct((rows, 128), jnp.float32),
            jax.ShapeDtypeStruct((rows, HID), jnp.float32),
        ],
    )
    return f(x, wcat, bcat.reshape(1, -1))


def _ge_kernel(y_ref, w_ref, b_ref, o_ref):
    r = y_ref[...] @ w_ref[...] + b_ref[...]
    blk = r.shape[0]
    z = jnp.zeros((blk, 128 - HID), jnp.float32)
    o_ref[...] = jnp.concatenate([r, z], axis=1)


def _egc_ge(y, w, b, blk, out_rows):
    rows = y.shape[0]
    f = pl.pallas_call(
        _ge_kernel,
        grid=(rows // blk,),
        in_specs=[
            pl.BlockSpec((blk, HID), lambda i: (i, 0)),
            pl.BlockSpec((HID, HID), lambda i: (0, 0)),
            pl.BlockSpec((1, HID), lambda i: (0, 0)),
        ],
        out_specs=pl.BlockSpec((blk, 128), lambda i: (i, 0)),
        out_shape=jax.ShapeDtypeStruct((out_rows, 128), jnp.float32),
    )
    return f(y, w, b.reshape(1, -1))


def _post_x_kernel(x_ref, xu_ref, acc_ref, g_ref, b_ref, o_ref):
    acc = acc_ref[...]
    h = acc[:, 0:96] / (acc[:, 96:192] + 1e-6)
    xo = _ln_silu(xu_ref[...] + h, g_ref[...], b_ref[...])
    o_ref[...] = x_ref[...] + xo


def _egc_post_x(x, xu, acc, g, b, blk):
    rows = x.shape[0]
    f = pl.pallas_call(
        _post_x_kernel,
        grid=(rows // blk,),
        in_specs=[
            pl.BlockSpec((blk, HID), lambda i: (i, 0)),
            pl.BlockSpec((blk, HID), lambda i: (i, 0)),
            pl.BlockSpec((blk, 192), lambda i: (i, 0)),
            pl.BlockSpec((1, HID), lambda i: (0, 0)),
            pl.BlockSpec((1, HID), lambda i: (0, 0)),
        ],
        out_specs=pl.BlockSpec((blk, HID), lambda i: (i, 0)),
        out_shape=jax.ShapeDtypeStruct((rows, HID), jnp.float32),
    )
    return f(x, xu, acc, g.reshape(1, -1), b.reshape(1, -1))


def _post_y_kernel(y_ref, m_ref, g_ref, b_ref, o_ref):
    yo = _ln_silu(m_ref[...][:, 0:96], g_ref[...], b_ref[...])
    o_ref[...] = y_ref[...] + yo


def _egc_post_y(y, m_arr, g, b, blk):
    rows = y.shape[0]
    f = pl.pallas_call(
        _post_y_kernel,
        grid=(rows // blk,),
        in_specs=[
            pl.BlockSpec((blk, HID), lambda i: (i, 0)),
            pl.BlockSpec((blk, 128), lambda i: (i, 0)),
            pl.BlockSpec((1, HID), lambda i: (0, 0)),
            pl.BlockSpec((1, HID), lambda i: (0, 0)),
        ],
        out_specs=pl.BlockSpec((blk, HID), lambda i: (i, 0)),
        out_shape=jax.ShapeDtypeStruct((rows, HID), jnp.float32),
    )
    return f(y, m_arr, g.reshape(1, -1), b.reshape(1, -1))


def _readout_kernel(x_ref, w_ref, b_ref, o_ref):
    o_ref[...] = x_ref[...] @ w_ref[...] + b_ref[...]


def _readout(p, x, blk):
    rows = x.shape[0]
    f = pl.pallas_call(
        _readout_kernel,
        grid=(rows // blk,),
        in_specs=[
            pl.BlockSpec((blk, HID), lambda i: (i, 0)),
            pl.BlockSpec((HID, 1), lambda i: (0, 0)),
            pl.BlockSpec((1, 1), lambda i: (0, 0)),
        ],
        out_specs=pl.BlockSpec((blk, 1), lambda i: (i, 0)),
        out_shape=jax.ShapeDtypeStruct((rows, 1), jnp.float32),
    )
    return f(x, p["w"], p["b"].reshape(1, 1))


def _edge_phase(esrc, edst, bh, ge, src, dst, nseg):
    m = esrc[:, :96][src] + edst[:, :96][dst] + ge[:src.shape[0], :96]
    sigma = jax.nn.sigmoid(m)
    ssh = jax.ops.segment_sum(sigma * bh[:, :96][src], dst, num_segments=nseg)
    ss = jax.ops.segment_sum(sigma, dst, num_segments=nseg)
    acc = jnp.concatenate([ssh, ss], axis=1)
    mpad = jnp.concatenate([m, jnp.zeros((m.shape[0], 32), jnp.float32)], axis=1)
    return mpad, acc


def _egc_layer(p, tp, src, dst, nseg, x, y, blk_x, blk_y, ge_rows):
    wcat = jnp.concatenate([p["src_gate"]["w"], p["dst_gate"]["w"],
                            p["dst_update"]["w"], p["src_update"]["w"]], axis=1)
    bcat = jnp.concatenate([p["src_gate"]["b"] + tp, p["dst_gate"]["b"],
                            p["dst_update"]["b"], p["src_update"]["b"]])
    esrc, edst, bh, xu = _egc_pre(x, wcat, bcat, blk_x)
    ge = _egc_ge(y, p["edge_gate"]["w"], p["edge_gate"]["b"], blk_y, ge_rows)
    m_arr, acc = _edge_phase(esrc, edst, bh, ge, src, dst, nseg)
    x_new = _egc_post_x(x, xu, acc, p["ln_n"]["g"], p["ln_n"]["b"], blk_x)
    y_new = _egc_post_y(y, m_arr, p["ln_e"]["g"], p["ln_e"]["b"], blk_y)
    return x_new, y_new


def kernel(edge_index, lg_edge_index, atom_feats, bondlength, cos_angles, timesteps, params):
    src, dst = edge_index[0], edge_index[1]
    lsrc, ldst = lg_edge_index[0], lg_edge_index[1]
    n = atom_feats.shape[0]
    e = bondlength.shape[0]

    egc_ps = ([lp["node"] for lp in params["alignn"]]
              + [lp["edge"] for lp in params["alignn"]]
              + list(params["gcn"])
              + [params["edges_l1"], params["edges_l2"], params["atoms_l"]])
    wp_all = jnp.concatenate([q["time_proj"]["w"] for q in egc_ps], axis=1)
    bp_all = jnp.concatenate([q["time_proj"]["b"] for q in egc_ps])
    tp_all = _time_tp(timesteps, params, len(egc_ps), wp_all, bp_all)

    x = _atom_emb(atom_feats, params["atom_emb"])
    y = _emb2(bondlength, params["edge_emb"][0], params["edge_emb"][1], 0.0, 8.0, 80, BLK_E)
    z = _emb2(cos_angles, params["angle_emb"][0], params["angle_emb"][1], -1.0, 1.0, 40, BLK_E)

    na = len(params["alignn"])
    for i, lp in enumerate(params["alignn"]):
        x, m = _egc_layer(lp["node"], tp_all[i], src, dst, n, x, y, BLK_N, BLK_E, e)
        y, z = _egc_layer(lp["edge"], tp_all[na + i], lsrc, ldst, e, m, z, BLK_E, BLK_E, e)
    for j, lp in enumerate(params["gcn"]):
        x, y = _egc_layer(lp, tp_all[2 * na + j], src, dst, n, x, y, BLK_N, BLK_E, e)
    xe, ye = _egc_layer(params["edges_l1"], tp_all[9], src, dst, n, x, y, BLK_N, BLK_E, e)
    xe, ye = _egc_layer(params["edges_l2"], tp_all[10], src, dst, n, xe, ye, BLK_N, BLK_E, e)
    edge_out = _readout(params["edges_ro"], ye, BLK_E)
    xa, ya = _egc_layer(params["atoms_l"], tp_all[11], src, dst, n, x, y, BLK_N, BLK_E, e)
    atom_out = _readout(params["atoms_ro"], xa, BLK_N)
    return jnp.concatenate([atom_out, edge_out], axis=0)
